# baseline (speedup 1.0000x reference)
import functools

import jax
import jax.numpy as jnp
from jax import lax
from jax.experimental import pallas as pl
from jax.experimental.pallas import tpu as pltpu
from jax.experimental.pallas import tpu_sc as plsc

_LANES = 16
_CHUNK = 128
_GW = 4 * _CHUNK
_GCH = 64


@functools.lru_cache(maxsize=None)
def _build_fused_kernel(B, V, D):
    info = plsc.get_sparse_core_info()
    NC, NS = info.num_cores, info.num_subcores
    DH = D // NC
    n_grp = V // _GW
    n_extra = (V - n_grp * _GW) // _CHUNK
    v_tail = V - n_grp * _GW - n_extra * _CHUNK
    max_g = 2 * ((n_grp + 2 * NS - 1) // (2 * NS))
    b_per_s = B // NS
    n_sub = b_per_s // _GCH
    rpp = _CHUNK // 4
    rows_half = V // 4

    mesh = plsc.VectorSubcoreMesh(core_axis_name="c", subcore_axis_name="s")

    @functools.partial(
        pl.kernel,
        mesh=mesh,
        out_type=(
            jax.ShapeDtypeStruct((D, B), jnp.float32),
            jax.ShapeDtypeStruct((NC * rows_half, 2 * D), jnp.float32),
        ),
        scratch_types=[
            pltpu.VMEM((b_per_s + _LANES,), jnp.int32),
            pltpu.VMEM((b_per_s,), jnp.int32),
            pltpu.VMEM((2, DH, _GW), jnp.float32),
            pltpu.VMEM((4, rpp, 2 * D + 1), jnp.float32),
            pltpu.VMEM((2, _GCH, 2 * D), jnp.float32),
            pltpu.VMEM((DH, b_per_s + 1), jnp.float32),
            pltpu.SemaphoreType.DMA,
            pltpu.SemaphoreType.DMA,
            pltpu.SemaphoreType.DMA,
        ],
        compiler_params=pltpu.CompilerParams(needs_layout_passes=False),
    )
    def emb(
        idx_hbm,
        table_t_hbm,
        tail_hbm,
        out_hbm,
        scr_hbm,
        idx_v,
        idx4_v,
        in_v,
        pan_v,
        rows_v,
        rows_t,
        sem_in,
        sem_out,
        sem_g,
    ):
        cid = lax.axis_index("c")
        sid = lax.axis_index("s")
        lane = lax.iota(jnp.int32, _LANES)
        d_base = pl.multiple_of(cid * DH, DH)
        row_base = cid * rows_half

        def group_id(g):
            gg = sid + g * NS
            return jnp.where(gg < n_grp, gg, sid)

        def in_window(g, buf):
            off = pl.multiple_of(group_id(g) * _GW, _GW)
            return (
                table_t_hbm.at[pl.ds(d_base, DH), pl.ds(off, _GW)],
                in_v.at[buf, :, pl.ds(0, _GW)],
            )

        def out_window(g, j):
            off = pl.multiple_of(
                row_base + (group_id(g) * 4 + j) * rpp, 8
            )
            return (
                pan_v.at[j, :, pl.ds(0, 2 * D)],
                scr_hbm.at[pl.ds(off, rpp)],
            )

        def start_in(g, buf):
            s, d = in_window(g, buf)
            pltpu.async_copy(s, d, sem_in)

        def wait_in(g, buf):
            s, d = in_window(g, buf)
            pltpu.make_async_copy(s, d, sem_in).wait()

        def start_out(g, j):
            s, d = out_window(g, j)
            pltpu.async_copy(s, d, sem_out)

        def wait_out(g, j):
            s, d = out_window(g, j)
            pltpu.make_async_copy(s, d, sem_out).wait()

        q_vecs = [(e0 + lane) >> 2 for e0 in range(0, _CHUNK, _LANES)]
        cb_vecs = [
            ((e0 + lane) & 3) * DH for e0 in range(0, _CHUNK, _LANES)
        ]

        def transpose_subpanel(buf, j, pan_buf):
            @plsc.parallel_loop(0, DH)
            def td(d):
                for e in range(_CHUNK // _LANES):
                    vals = in_v[
                        buf, d, pl.ds(j * _CHUNK + e * _LANES, _LANES)
                    ]
                    plsc.store_scatter(
                        pan_v.at[pan_buf], [q_vecs[e], cb_vecs[e] + d], vals
                    )

        def do_group(g, buf, first):
            wait_in(g, buf)
            for j in range(4):
                if not first:
                    wait_out(g - 1, j)
                transpose_subpanel(buf, j, j)
                start_out(g, j)
            start_in(g + 2, buf)

        start_in(0, 0)
        start_in(1, 1)
        do_group(0, 0, True)
        do_group(1, 1, False)

        for h in range(1, max_g // 2):
            do_group(2 * h, 0, False)
            do_group(2 * h + 1, 1, False)
        for j in range(4):
            wait_out(max_g - 1, j)
        wait_in(max_g, 0)
        wait_in(max_g + 1, 1)

        for e in range(n_extra):

            @pl.when(sid == 1)
            def _():
                p = n_grp * 4 + e
                pltpu.sync_copy(
                    table_t_hbm.at[
                        pl.ds(d_base, DH), pl.ds(p * _CHUNK, _CHUNK)
                    ],
                    in_v.at[0, :, pl.ds(0, _CHUNK)],
                )
                transpose_subpanel(0, 0, 0)
                pltpu.sync_copy(
                    pan_v.at[0, :, pl.ds(0, 2 * D)],
                    scr_hbm.at[pl.ds(row_base + p * rpp, rpp)],
                )

        if v_tail:

            @pl.when(sid == 0)
            def _():
                pltpu.sync_copy(
                    tail_hbm.at[cid],
                    scr_hbm.at[
                        pl.ds(
                            row_base + (n_grp * 4 + n_extra) * rpp,
                            v_tail // 4,
                        )
                    ],
                )

        plsc.subcore_barrier()

        base = sid * b_per_s
        pltpu.sync_copy(
            idx_hbm.at[pl.ds(base, b_per_s)], idx_v.at[pl.ds(0, b_per_s)]
        )

        @plsc.parallel_loop(0, b_per_s // _LANES)
        def mk_idx(g):
            sl = pl.ds(g * _LANES, _LANES)
            idx4_v[sl] = (idx_v[sl] >> 2) + row_base

        d_vecs = [d0 * _LANES + lane for d0 in range(DH // _LANES)]

        def start_gather(j):
            return pltpu.async_copy(
                scr_hbm.at[idx4_v.at[pl.ds(j * _GCH, _GCH)]],
                rows_v.at[j % 2],
                sem_g,
            )

        pending = start_gather(0)
        for j in range(n_sub):
            nxt = start_gather(j + 1) if j + 1 < n_sub else None
            pending.wait()
            buf = j % 2

            @plsc.parallel_loop(0, _GCH)
            def tb(b):
                gb = j * _GCH + b
                sub = idx_v[pl.ds(gb, _LANES)][0] & 3
                b_vec = jnp.full((_LANES,), gb, jnp.int32)
                for d0 in range(DH // _LANES):
                    vals = rows_v[
                        buf, b, pl.ds(sub * DH + d0 * _LANES, _LANES)
                    ]
                    plsc.store_scatter(rows_t, [d_vecs[d0], b_vec], vals)

            pending = nxt

        pltpu.sync_copy(
            rows_t.at[:, pl.ds(0, b_per_s)],
            out_hbm.at[pl.ds(d_base, DH), pl.ds(base, b_per_s)],
        )

    return emb


def kernel(disaster_type_idx, embedding_weight):
    (B,) = disaster_type_idx.shape
    V, D = embedding_weight.shape
    emb = _build_fused_kernel(B, V, D)
    n128 = V // 128
    v_tail = V - n128 * 128
    tail = (
        embedding_weight[n128 * 128 :, :]
        .reshape(v_tail // 4, 4, 2, D // 2)
        .transpose(2, 0, 1, 3)
        .reshape(2, v_tail // 4, 2 * D)
    )
    out_t, _ = emb(
        disaster_type_idx.astype(jnp.int32), embedding_weight.T, tail
    )
    return out_t.T


# speedup vs baseline: 1.6886x; 1.6886x over previous
import functools

import jax
import jax.numpy as jnp
from jax import lax
from jax.experimental import pallas as pl
from jax.experimental.pallas import tpu as pltpu
from jax.experimental.pallas import tpu_sc as plsc

_CHUNK = 128
_LANES = 16


@functools.lru_cache(maxsize=None)
def _build_emb_kernel(B, V, D):
    info = plsc.get_sparse_core_info()
    num_workers = info.num_cores * info.num_subcores
    b_per_w = B // num_workers
    n_chunks = b_per_w // _CHUNK

    mesh = plsc.VectorSubcoreMesh(core_axis_name="c", subcore_axis_name="s")

    @functools.partial(
        pl.kernel,
        mesh=mesh,
        out_type=jax.ShapeDtypeStruct((D, B), jnp.float32),
        scratch_types=[
            pltpu.VMEM((b_per_w,), jnp.int32),
            pltpu.VMEM((b_per_w, D), jnp.float32),
            pltpu.VMEM((D, b_per_w + 1), jnp.float32),
            pltpu.SemaphoreType.DMA,
            pltpu.SemaphoreType.DMA,
        ],
        compiler_params=pltpu.CompilerParams(
            use_tc_tiling_on_sc=False, needs_layout_passes=False
        ),
    )
    def emb(idx_hbm, table_hbm, out_hbm, idx_v, rows_v, rows_t, sem, out_sem):
        wid = lax.axis_index("s") * info.num_cores + lax.axis_index("c")
        base = wid * b_per_w
        pltpu.sync_copy(idx_hbm.at[pl.ds(base, b_per_w)], idx_v)

        def start_gather(j):
            return pltpu.async_copy(
                table_hbm.at[idx_v.at[pl.ds(j * _CHUNK, _CHUNK)]],
                rows_v.at[pl.ds(j * _CHUNK, _CHUNK)],
                sem,
            )

        lane = lax.iota(jnp.int32, _LANES)
        d_vecs = [d0 * _LANES + lane for d0 in range(D // _LANES)]

        pending = start_gather(0)
        out_copies = []
        for j in range(n_chunks):
            nxt = start_gather(j + 1) if j + 1 < n_chunks else None
            pending.wait()

            @plsc.parallel_loop(j * _CHUNK, (j + 1) * _CHUNK)
            def transpose_row(b):
                b_vec = jnp.full((_LANES,), b, jnp.int32)
                for d0 in range(D // _LANES):
                    vals = rows_v[b, pl.ds(d0 * _LANES, _LANES)]
                    plsc.store_scatter(rows_t, [d_vecs[d0], b_vec], vals)

            out_copies.append(
                pltpu.async_copy(
                    rows_t.at[:, pl.ds(j * _CHUNK, _CHUNK)],
                    out_hbm.at[:, pl.ds(base + j * _CHUNK, _CHUNK)],
                    out_sem,
                )
            )
            pending = nxt
        for c in out_copies:
            c.wait()

    return emb


def kernel(disaster_type_idx, embedding_weight):
    (B,) = disaster_type_idx.shape
    V, D = embedding_weight.shape
    emb = _build_emb_kernel(B, V, D)
    out_t = emb(disaster_type_idx.astype(jnp.int32), embedding_weight)
    return out_t.T


# speedup vs baseline: 2.4336x; 1.4412x over previous
import functools

import jax
import jax.numpy as jnp
from jax import lax
from jax.experimental import pallas as pl
from jax.experimental.pallas import tpu as pltpu
from jax.experimental.pallas import tpu_sc as plsc

_LANES = 16
_CHUNK = 128
_GW = 4 * _CHUNK
_GCH = 64


@functools.lru_cache(maxsize=None)
def _build_fused_kernel(B, V, D):
    info = plsc.get_sparse_core_info()
    NC, NS = info.num_cores, info.num_subcores
    DH = D // NC
    n_grp = V // _GW
    n_extra = (V - n_grp * _GW) // _CHUNK
    v_tail = V - n_grp * _GW - n_extra * _CHUNK
    max_g = 2 * ((n_grp + 2 * NS - 1) // (2 * NS))
    b_per_s = B // NS
    n_sub = b_per_s // _GCH
    rpp = _CHUNK // 4
    rows_half = V // 4

    mesh = plsc.VectorSubcoreMesh(core_axis_name="c", subcore_axis_name="s")

    @functools.partial(
        pl.kernel,
        mesh=mesh,
        out_type=(
            jax.ShapeDtypeStruct((D, B), jnp.float32),
            jax.ShapeDtypeStruct((NC * rows_half, 2 * D), jnp.float32),
        ),
        scratch_types=[
            pltpu.VMEM((b_per_s + _LANES,), jnp.int32),
            pltpu.VMEM((b_per_s,), jnp.int32),
            pltpu.VMEM((2, DH, _GW + 1), jnp.float32),
            pltpu.VMEM((4, rpp, 2 * D), jnp.float32),
            pltpu.VMEM((2, _GCH, 2 * D), jnp.float32),
            pltpu.VMEM((DH, b_per_s + 1), jnp.float32),
            pltpu.SemaphoreType.DMA,
            pltpu.SemaphoreType.DMA,
            pltpu.SemaphoreType.DMA,
        ],
        compiler_params=pltpu.CompilerParams(needs_layout_passes=False),
    )
    def emb(
        idx_hbm,
        table_t_hbm,
        tail_hbm,
        out_hbm,
        scr_hbm,
        idx_v,
        idx4_v,
        in_v,
        pan_v,
        rows_v,
        rows_t,
        sem_in,
        sem_out,
        sem_g,
    ):
        cid = lax.axis_index("c")
        sid = lax.axis_index("s")
        lane = lax.iota(jnp.int32, _LANES)
        d_base = pl.multiple_of(cid * DH, DH)
        row_base = cid * rows_half

        def group_id(g):
            gg = sid + g * NS
            return jnp.where(gg < n_grp, gg, sid)

        def in_window(g, buf):
            off = pl.multiple_of(group_id(g) * _GW, _GW)
            return (
                table_t_hbm.at[pl.ds(d_base, DH), pl.ds(off, _GW)],
                in_v.at[buf, :, pl.ds(0, _GW)],
            )

        def out_window(g, j):
            off = pl.multiple_of(row_base + (group_id(g) * 4 + j) * rpp, 8)
            return (pan_v.at[j], scr_hbm.at[pl.ds(off, rpp)])

        def start_in(g, buf):
            s, d = in_window(g, buf)
            pltpu.async_copy(s, d, sem_in)

        def wait_in(g, buf):
            s, d = in_window(g, buf)
            pltpu.make_async_copy(s, d, sem_in).wait()

        def start_out(g, j):
            s, d = out_window(g, j)
            pltpu.async_copy(s, d, sem_out)

        def wait_out(g, j):
            s, d = out_window(g, j)
            pltpu.make_async_copy(s, d, sem_out).wait()

        def transpose_subpanel(buf, j, pan_buf, width=_CHUNK):
            @functools.partial(plsc.parallel_loop, 0, width // 4, unroll=8)
            def tq(q):
                base_col = jnp.full(
                    (_LANES,), j * _CHUNK + 4 * q, jnp.int32
                )
                for k in range(4):
                    col = base_col + k
                    for g2 in range(DH // _LANES):
                        vals = plsc.load_gather(
                            in_v.at[buf], [g2 * _LANES + lane, col]
                        )
                        pan_v[
                            pan_buf, q, pl.ds(k * DH + g2 * _LANES, _LANES)
                        ] = vals

        def do_group(g, buf, first):
            wait_in(g, buf)
            for j in range(4):
                if not first:
                    wait_out(g - 1, j)
                transpose_subpanel(buf, j, j)
                start_out(g, j)
            start_in(g + 2, buf)

        start_in(0, 0)
        start_in(1, 1)
        do_group(0, 0, True)
        do_group(1, 1, False)

        def body(h, _):
            do_group(2 * h, 0, False)
            do_group(2 * h + 1, 1, False)
            return 0

        lax.fori_loop(1, max_g // 2, body, 0)
        for j in range(4):
            wait_out(max_g - 1, j)
        wait_in(max_g, 0)
        wait_in(max_g + 1, 1)

        for e in range(n_extra):

            @pl.when(sid == 1)
            def _():
                p = n_grp * 4 + e
                pltpu.sync_copy(
                    table_t_hbm.at[
                        pl.ds(d_base, DH), pl.ds(p * _CHUNK, _CHUNK)
                    ],
                    in_v.at[0, :, pl.ds(0, _CHUNK)],
                )
                transpose_subpanel(0, 0, 0)
                pltpu.sync_copy(
                    pan_v.at[0],
                    scr_hbm.at[pl.ds(row_base + p * rpp, rpp)],
                )

        if v_tail:

            @pl.when(sid == 0)
            def _():
                pltpu.sync_copy(
                    tail_hbm.at[cid],
                    scr_hbm.at[
                        pl.ds(
                            row_base + (n_grp * 4 + n_extra) * rpp,
                            v_tail // 4,
                        )
                    ],
                )

        plsc.subcore_barrier()

        base = sid * b_per_s
        pltpu.sync_copy(
            idx_hbm.at[pl.ds(base, b_per_s)], idx_v.at[pl.ds(0, b_per_s)]
        )

        @plsc.parallel_loop(0, b_per_s // _LANES)
        def mk_idx(g):
            sl = pl.ds(g * _LANES, _LANES)
            idx4_v[sl] = (idx_v[sl] >> 2) + row_base

        d_vecs = [d0 * _LANES + lane for d0 in range(DH // _LANES)]

        def start_gather(j):
            return pltpu.async_copy(
                scr_hbm.at[idx4_v.at[pl.ds(j * _GCH, _GCH)]],
                rows_v.at[j % 2],
                sem_g,
            )

        pending = start_gather(0)
        for j in range(n_sub):
            nxt = start_gather(j + 1) if j + 1 < n_sub else None
            pending.wait()
            buf = j % 2

            @plsc.parallel_loop(0, _GCH)
            def tb(b):
                gb = j * _GCH + b
                sub = idx_v[pl.ds(gb, _LANES)][0] & 3
                b_vec = jnp.full((_LANES,), gb, jnp.int32)
                for d0 in range(DH // _LANES):
                    vals = rows_v[
                        buf, b, pl.ds(sub * DH + d0 * _LANES, _LANES)
                    ]
                    plsc.store_scatter(rows_t, [d_vecs[d0], b_vec], vals)

            pending = nxt

        pltpu.sync_copy(
            rows_t.at[:, pl.ds(0, b_per_s)],
            out_hbm.at[pl.ds(d_base, DH), pl.ds(base, b_per_s)],
        )

    return emb


def kernel(disaster_type_idx, embedding_weight):
    (B,) = disaster_type_idx.shape
    V, D = embedding_weight.shape
    emb = _build_fused_kernel(B, V, D)
    n128 = V // 128
    v_tail = V - n128 * 128
    tail = (
        embedding_weight[n128 * 128 :, :]
        .reshape(v_tail // 4, 4, 2, D // 2)
        .transpose(2, 0, 1, 3)
        .reshape(2, v_tail // 4, 2 * D)
    )
    out_t, _ = emb(
        disaster_type_idx.astype(jnp.int32), embedding_weight.T, tail
    )
    return out_t.T
